# baseline (speedup 1.0000x reference)
import functools

import jax
import jax.numpy as jnp
from jax import lax
from jax.experimental import pallas as pl
from jax.experimental.pallas import tpu as pltpu
from jax.experimental.pallas import tpu_sc as plsc

B = 16384
R = 64
NC = 2
NS = 16
NW = NC * NS
BPW = B // NW
CH = 128
NCH = BPW // CH
GROUPS = BPW // 16


def _transpose_tc_body(x_ref, o_ref):
    o_ref[:] = x_ref[:].T


def _cpd_sc_body(idx_hbm, e0_hbm, e1_hbm, e2_hbm, out_hbm,
                 idx_v, r0, r1, r2, out_v, sem):
    wid = lax.axis_index("s") * NC + lax.axis_index("c")
    base = wid * BPW

    for m in range(3):
        pltpu.sync_copy(idx_hbm.at[m, pl.ds(base, BPW)], idx_v.at[m])

    copies = []
    for m, (tab, dst) in enumerate(((e0_hbm, r0), (e1_hbm, r1), (e2_hbm, r2))):
        for j in range(NCH):
            copies.append(
                pltpu.async_copy(tab.at[idx_v.at[m, pl.ds(j * CH, CH)]],
                                 dst.at[pl.ds(j * CH, CH)], sem))
    for cp in copies:
        cp.wait()

    def group(g, carry):
        b0 = g * 16
        for rr in range(16):
            row = b0 + rr
            acc = None
            for c in range(4):
                a = r0[row, pl.ds(c * 16, 16)]
                bb = r1[row, pl.ds(c * 16, 16)]
                d = r2[row, pl.ds(c * 16, 16)]
                p = a * bb * d
                acc = p if acc is None else acc + p
            out_v[pl.ds(row * 16, 16)] = acc
        return carry

    lax.fori_loop(0, GROUPS, group, 0)

    pltpu.sync_copy(out_v, out_hbm.at[pl.ds(wid * BPW * 16, BPW * 16)])


def _reduce_tc_body(x_ref, o_ref):
    x = x_ref[:].reshape(256, 128)
    l_i = lax.broadcasted_iota(jnp.int32, (128, 128), 0)
    j_i = lax.broadcasted_iota(jnp.int32, (128, 128), 1)
    wa = ((j_i < 64) & (l_i // 2 == j_i)).astype(jnp.float32)
    wb = ((j_i >= 64) & (l_i // 2 == j_i - 64)).astype(jnp.float32)
    n = 256
    for _ in range(4):
        h = n // 2
        r_i = lax.broadcasted_iota(jnp.int32, (h, n), 0)
        c_i = lax.broadcasted_iota(jnp.int32, (h, n), 1)
        ae = (c_i == 2 * r_i).astype(jnp.float32)
        ao = (c_i == 2 * r_i + 1).astype(jnp.float32)
        xe = jnp.dot(ae, x, preferred_element_type=jnp.float32)
        xo = jnp.dot(ao, x, preferred_element_type=jnp.float32)
        x = (jnp.dot(xe, wa, preferred_element_type=jnp.float32)
             + jnp.dot(xo, wb, preferred_element_type=jnp.float32))
        n = h
    o_ref[:] = x.reshape(2048)


@jax.jit
def kernel(idxs, E0, E1, E2):
    idxs = idxs.astype(jnp.int32)

    tr_rows = 2048
    idx_t = pl.pallas_call(
        _transpose_tc_body,
        grid=(B // tr_rows,),
        in_specs=[pl.BlockSpec((tr_rows, 3), lambda i: (i, 0))],
        out_specs=pl.BlockSpec((3, tr_rows), lambda i: (0, i)),
        out_shape=jax.ShapeDtypeStruct((3, B), jnp.int32),
    )(idxs)

    mesh = plsc.VectorSubcoreMesh(core_axis_name="c", subcore_axis_name="s")
    sc_fn = pl.kernel(
        _cpd_sc_body,
        mesh=mesh,
        out_type=jax.ShapeDtypeStruct((B * 16,), jnp.float32),
        scratch_types=[
            pltpu.VMEM((3, BPW), jnp.int32),
            pltpu.VMEM((BPW, R), jnp.float32),
            pltpu.VMEM((BPW, R), jnp.float32),
            pltpu.VMEM((BPW, R), jnp.float32),
            pltpu.VMEM((BPW * 16,), jnp.float32),
            pltpu.SemaphoreType.DMA,
        ],
        compiler_params=pltpu.CompilerParams(use_tc_tiling_on_sc=False),
    )
    partials = sc_fn(idx_t, E0, E1, E2)

    red_rows = 2048
    out = pl.pallas_call(
        _reduce_tc_body,
        grid=(B // red_rows,),
        in_specs=[pl.BlockSpec((red_rows * 16,), lambda i: (i,))],
        out_specs=pl.BlockSpec((red_rows,), lambda i: (i,)),
        out_shape=jax.ShapeDtypeStruct((B,), jnp.float32),
    )(partials)
    return out
